# baseline (speedup 1.0000x reference)
import jax
import jax.numpy as jnp
from jax import lax
from jax.experimental import pallas as pl
from jax.experimental.pallas import tpu as pltpu

_ROWS = 32
_COLS = 1_000_000
_S = 800
_L = 1250
_CH = 16
_NCH = _S // _CH

_ROT_A = (13, 15, 26, 6)
_ROT_B = (17, 29, 16, 24)
_KS = (0, 1, 0x1BD11BDA ^ 0 ^ 1)


def _threefry_bits(x1):
    x0 = x1
    x1 = ((x1 << 13) | (x1 >> 19)) ^ x0
    first = True
    for i in range(5):
        rots = _ROT_A if i % 2 == 0 else _ROT_B
        for r in (rots[1:] if first else rots):
            x0 = x0 + x1
            x1 = (x1 << r) | (x1 >> (32 - r))
            x1 = x1 ^ x0
        first = False
        x0 = x0 + jnp.uint32(_KS[(i + 1) % 3])
        x1 = x1 + jnp.uint32(_KS[(i + 2) % 3] + i + 1)
    return x0 ^ x1


def _gumbel_softmax_kernel(inv_t_ref, x_ref, o_ref):
    row = pl.program_id(0)
    inv_t = inv_t_ref[0, 0]
    eps = jnp.float32(1e-20)
    sub = lax.broadcasted_iota(jnp.uint32, (_CH, _L), 0)
    lane = lax.broadcasted_iota(jnp.uint32, (_CH, _L), 1)
    cvec = sub * jnp.uint32(_L) + lane
    base = jnp.uint32(row * _COLS + 1)

    def z_body(k, m_vec):
        off = (k * (_CH * _L)).astype(jnp.uint32) + base
        bits = _threefry_bits(cvec + off)
        fbits = (bits >> 9) | jnp.uint32(0x3F800000)
        u = lax.bitcast_convert_type(fbits, jnp.float32) - jnp.float32(1.0)
        g = -jnp.log(-jnp.log(u + eps) + eps)
        z = (x_ref[0, pl.ds(k * _CH, _CH), :] + g) * inv_t
        o_ref[0, pl.ds(k * _CH, _CH), :] = z
        return jnp.maximum(m_vec, z)

    m_vec = lax.fori_loop(
        0, _NCH, z_body, jnp.full((_CH, _L), -jnp.inf, jnp.float32))
    m = jnp.max(m_vec)

    def e_body(k, s_vec):
        e = jnp.exp(o_ref[0, pl.ds(k * _CH, _CH), :] - m)
        o_ref[0, pl.ds(k * _CH, _CH), :] = e
        return s_vec + e

    s_vec = lax.fori_loop(
        0, _NCH, e_body, jnp.zeros((_CH, _L), jnp.float32))
    inv_s = jnp.float32(1.0) / jnp.sum(s_vec)

    def scale_body(k, carry):
        o_ref[0, pl.ds(k * _CH, _CH), :] *= inv_s
        return carry

    lax.fori_loop(0, _NCH, scale_body, jnp.float32(0.0))


def kernel(logits, temperature, use_gpu):
    del use_gpu
    inv_t = (jnp.float32(1.0)
             / jnp.asarray(temperature, jnp.float32)).reshape(1, 1)
    out = pl.pallas_call(
        _gumbel_softmax_kernel,
        grid=(_ROWS,),
        in_specs=[
            pl.BlockSpec(memory_space=pltpu.SMEM),
            pl.BlockSpec((1, _S, _L), lambda i: (i, 0, 0)),
        ],
        out_specs=pl.BlockSpec((1, _S, _L), lambda i: (i, 0, 0)),
        out_shape=jax.ShapeDtypeStruct((_ROWS, _S, _L), jnp.float32),
        compiler_params=pltpu.CompilerParams(
            dimension_semantics=("parallel",),
        ),
    )(inv_t, logits.reshape(_ROWS, _S, _L))
    return out.reshape(_ROWS, _COLS)


# speedup vs baseline: 1.0450x; 1.0450x over previous
import jax
import jax.numpy as jnp
from jax import lax
from jax.experimental import pallas as pl
from jax.experimental.pallas import tpu as pltpu

_ROWS = 32
_COLS = 1_000_000
_S = 800
_L = 1250
_CH = 40
_NCH = _S // _CH

_ROT_A = (13, 15, 26, 6)
_ROT_B = (17, 29, 16, 24)
_KS = (0, 1, 0x1BD11BDA ^ 0 ^ 1)


def _threefry_bits(x1):
    x0 = x1
    x1 = ((x1 << 13) | (x1 >> 19)) ^ x0
    first = True
    for i in range(5):
        rots = _ROT_A if i % 2 == 0 else _ROT_B
        for r in (rots[1:] if first else rots):
            x0 = x0 + x1
            x1 = (x1 << r) | (x1 >> (32 - r))
            x1 = x1 ^ x0
        first = False
        x0 = x0 + jnp.uint32(_KS[(i + 1) % 3])
        x1 = x1 + jnp.uint32(_KS[(i + 2) % 3] + i + 1)
    return x0 ^ x1


def _gumbel_softmax_kernel(inv_t_ref, x_ref, o_ref):
    row = pl.program_id(0)
    inv_t = inv_t_ref[0, 0]
    eps = jnp.float32(1e-20)
    sub = lax.broadcasted_iota(jnp.uint32, (_CH, _L), 0)
    lane = lax.broadcasted_iota(jnp.uint32, (_CH, _L), 1)
    cvec = sub * jnp.uint32(_L) + lane
    base = jnp.uint32(row * _COLS + 1)

    def z_body(k, m_vec):
        off = (k * (_CH * _L)).astype(jnp.uint32) + base
        bits = _threefry_bits(cvec + off)
        fbits = (bits >> 9) | jnp.uint32(0x3F800000)
        u = lax.bitcast_convert_type(fbits, jnp.float32) - jnp.float32(1.0)
        g = -jnp.log(-jnp.log(u + eps) + eps)
        z = (x_ref[0, pl.ds(k * _CH, _CH), :] + g) * inv_t
        o_ref[0, pl.ds(k * _CH, _CH), :] = z
        return jnp.maximum(m_vec, z)

    m_vec = lax.fori_loop(
        0, _NCH, z_body, jnp.full((_CH, _L), -jnp.inf, jnp.float32))
    m = jnp.max(m_vec)

    def e_body(k, s_vec):
        e = jnp.exp(o_ref[0, pl.ds(k * _CH, _CH), :] - m)
        o_ref[0, pl.ds(k * _CH, _CH), :] = e
        return s_vec + e

    s_vec = lax.fori_loop(
        0, _NCH, e_body, jnp.zeros((_CH, _L), jnp.float32))
    inv_s = jnp.float32(1.0) / jnp.sum(s_vec)

    def scale_body(k, carry):
        o_ref[0, pl.ds(k * _CH, _CH), :] *= inv_s
        return carry

    lax.fori_loop(0, _NCH, scale_body, jnp.float32(0.0))


def kernel(logits, temperature, use_gpu):
    del use_gpu
    inv_t = (jnp.float32(1.0)
             / jnp.asarray(temperature, jnp.float32)).reshape(1, 1)
    out = pl.pallas_call(
        _gumbel_softmax_kernel,
        grid=(_ROWS,),
        in_specs=[
            pl.BlockSpec(memory_space=pltpu.SMEM),
            pl.BlockSpec((1, _S, _L), lambda i: (i, 0, 0)),
        ],
        out_specs=pl.BlockSpec((1, _S, _L), lambda i: (i, 0, 0)),
        out_shape=jax.ShapeDtypeStruct((_ROWS, _S, _L), jnp.float32),
        compiler_params=pltpu.CompilerParams(
            dimension_semantics=("parallel",),
        ),
    )(inv_t, logits.reshape(_ROWS, _S, _L))
    return out.reshape(_ROWS, _COLS)


# speedup vs baseline: 1.0941x; 1.0469x over previous
import jax
import jax.numpy as jnp
from jax import lax
from jax.experimental import pallas as pl
from jax.experimental.pallas import tpu as pltpu

_ROWS = 32
_COLS = 1_000_000
_S = 1000
_L = 1000
_CZ = 8
_UZ = 5
_NZ = _S // (_CZ * _UZ)
_CE = 40
_NE = _S // _CE

_ROT_A = (13, 15, 26, 6)
_ROT_B = (17, 29, 16, 24)
_KS = (0, 1, 0x1BD11BDA ^ 0 ^ 1)


def _threefry_bits(x1):
    x0 = x1
    x1 = ((x1 << 13) | (x1 >> 19)) ^ x0
    first = True
    for i in range(5):
        rots = _ROT_A if i % 2 == 0 else _ROT_B
        for r in (rots[1:] if first else rots):
            x0 = x0 + x1
            x1 = (x1 << r) | (x1 >> (32 - r))
            x1 = x1 ^ x0
        first = False
        x0 = x0 + jnp.uint32(_KS[(i + 1) % 3])
        x1 = x1 + jnp.uint32(_KS[(i + 2) % 3] + i + 1)
    return x0 ^ x1


def _gumbel_softmax_kernel(inv_t_ref, x_ref, o_ref):
    row = pl.program_id(0)
    inv_t = inv_t_ref[0, 0]
    eps = jnp.float32(1e-20)
    sub = lax.broadcasted_iota(jnp.uint32, (_CZ, _L), 0)
    lane = lax.broadcasted_iota(jnp.uint32, (_CZ, _L), 1)
    cvec = sub * jnp.uint32(_L) + lane
    base = jnp.uint32(row * _COLS + 1)

    def z_body(k, m_vec):
        s0 = k * (_CZ * _UZ)
        for j in range(_UZ):
            off = (s0 + j * _CZ).astype(jnp.uint32) * jnp.uint32(_L) + base
            bits = _threefry_bits(cvec + off)
            fbits = (bits >> 9) | jnp.uint32(0x3F800000)
            u = lax.bitcast_convert_type(fbits, jnp.float32) - jnp.float32(1.0)
            g = -jnp.log(-jnp.log(u + eps) + eps)
            z = (x_ref[0, pl.ds(s0 + j * _CZ, _CZ), :] + g) * inv_t
            o_ref[0, pl.ds(s0 + j * _CZ, _CZ), :] = z
            m_vec = jnp.maximum(m_vec, z)
        return m_vec

    m_vec = lax.fori_loop(
        0, _NZ, z_body, jnp.full((_CZ, _L), -jnp.inf, jnp.float32))
    m = jnp.max(m_vec)

    def e_body(k, s_vec):
        e = jnp.exp(o_ref[0, pl.ds(k * _CE, _CE), :] - m)
        o_ref[0, pl.ds(k * _CE, _CE), :] = e
        return s_vec + e

    s_vec = lax.fori_loop(
        0, _NE, e_body, jnp.zeros((_CE, _L), jnp.float32))
    inv_s = jnp.float32(1.0) / jnp.sum(s_vec)

    def scale_body(k, carry):
        o_ref[0, pl.ds(k * _CE, _CE), :] *= inv_s
        return carry

    lax.fori_loop(0, _NE, scale_body, jnp.float32(0.0))


def kernel(logits, temperature, use_gpu):
    del use_gpu
    inv_t = (jnp.float32(1.0)
             / jnp.asarray(temperature, jnp.float32)).reshape(1, 1)
    out = pl.pallas_call(
        _gumbel_softmax_kernel,
        grid=(_ROWS,),
        in_specs=[
            pl.BlockSpec(memory_space=pltpu.SMEM),
            pl.BlockSpec((1, _S, _L), lambda i: (i, 0, 0)),
        ],
        out_specs=pl.BlockSpec((1, _S, _L), lambda i: (i, 0, 0)),
        out_shape=jax.ShapeDtypeStruct((_ROWS, _S, _L), jnp.float32),
        compiler_params=pltpu.CompilerParams(
            dimension_semantics=("parallel",),
        ),
    )(inv_t, logits.reshape(_ROWS, _S, _L))
    return out.reshape(_ROWS, _COLS)
